# baseline (device time: 45104 ns/iter reference)
import os as _os

import numpy as np
import jax
import jax.numpy as jnp
from jax import lax
from jax.experimental import pallas as pl
from jax.experimental.pallas import tpu as pltpu

N_DEV = 8
B = 2
SQ = 256
D = 768
HC = 4
DH = 64
CW = HC * DH
BSQ = B * SQ

_sem_signal = getattr(pl, "semaphore_signal", None) or pltpu.semaphore_signal
_sem_wait = getattr(pl, "semaphore_wait", None) or pltpu.semaphore_wait
_CompilerParams = getattr(pltpu, "CompilerParams", None) or getattr(
    pltpu, "TPUCompilerParams"
)

_COMPUTE_ONLY = _os.environ.get("SCBAND_COMPUTE_ONLY") == "1"


def _consts():
    inv = 1.0 / (10000.0 ** (np.arange(0, DH, 2) / DH))
    pos = np.arange(SQ)[:, None] * inv[None, :]
    cos = np.repeat(np.cos(pos), 2, axis=-1)
    sin = np.repeat(np.sin(pos), 2, axis=-1)
    a = np.sqrt(0.125)
    cosm = np.tile(cos * a, (B, HC)).astype(np.float32)
    sinm = np.tile(sin * a, (B, HC)).astype(np.float32)
    r = np.zeros((DH, DH), np.float32)
    for i in range(0, DH, 2):
        r[i + 1, i] = -1.0
        r[i, i + 1] = 1.0
    rot = np.kron(np.eye(HC, dtype=np.float32), r)
    return cosm, sinm, rot


_COS, _SIN, _ROT = _consts()


def kernel(x, Wq, Wk, Wv, Wo):
    bf16 = jnp.bfloat16
    f32 = jnp.float32
    i8 = jnp.int8

    def body(x_ref, wq_ref, wk_ref, wv_ref, wo_ref, cos_ref, sin_ref,
             rot_ref, out_ref, xb, wbuf, obuf, scb,
             s_r, r_l, s_l, r_r, s_z, r_z,
             t_r, u_l, t_l, u_r, t_z, u_z):
        me = lax.axis_index("i")
        base = (me // 4) * 4
        pp = me - base
        right = base + lax.rem(pp + 1, 4)
        left = base + lax.rem(pp + 3, 4)
        partner = lax.rem(me + 4, N_DEV)

        barrier = pltpu.get_barrier_semaphore()
        for nbr in (left, right, partner):
            _sem_signal(barrier, inc=1, device_id=(nbr,),
                        device_id_type=pl.DeviceIdType.MESH)
        _sem_wait(barrier, 3)

        cosm = cos_ref[...]
        sinm = sin_ref[...]
        rotm = rot_ref[...]

        xb[0:SQ, :] = x_ref[0].astype(bf16)
        xb[SQ:BSQ, :] = x_ref[1].astype(bf16)

        def quant(w, pair):
            a = jnp.max(jnp.abs(w), axis=0, keepdims=True)
            if pair:
                ab = a.astype(bf16)
                swap = jnp.dot(ab, jnp.abs(rotm),
                               preferred_element_type=f32).astype(bf16)
                a = jnp.maximum(ab, swap).astype(f32)
            a = jnp.maximum(a, 1e-20)
            qi = jnp.clip(jnp.round(w * (127.0 / a)),
                          -127.0, 127.0).astype(i8)
            return qi, a * (1.0 / 127.0)

        def stage_comp(c):
            if c < 3:
                ref, pair = ((wq_ref, True), (wk_ref, True),
                             (wv_ref, False))[c]
                qi, sc = quant(ref[...], pair)
                wbuf[0, c] = qi
                scb[0, c, :] = sc[0]
            else:
                w = wo_ref[...]
                a = jnp.maximum(jnp.max(jnp.abs(w), axis=1, keepdims=True),
                                1e-20)
                obuf[0] = jnp.clip(jnp.round(w * (127.0 / a)),
                                   -127.0, 127.0).astype(i8)
                scb[0, 3, :] = a[:, 0] * (1.0 / 127.0)

        def compute(slot, first=False):
            xv = xb[...]
            w3 = wbuf[slot]
            wo_i = obuf[slot]
            sq = scb[slot, 0, :]
            sk = scb[slot, 1, :]
            sv = scb[slot, 2, :]
            so = scb[slot, 3, :]
            q = jnp.dot(xv, w3[0].astype(bf16),
                        preferred_element_type=f32).astype(bf16)
            k = jnp.dot(xv, w3[1].astype(bf16),
                        preferred_element_type=f32).astype(bf16)
            vb = jnp.dot(xv, w3[2].astype(bf16),
                         preferred_element_type=f32).astype(bf16)
            vb = vb * sv.astype(bf16)[None, :]
            qr = q * cosm + jnp.dot(
                q, rotm, preferred_element_type=f32).astype(bf16) * sinm
            kr = k * cosm + jnp.dot(
                k, rotm, preferred_element_type=f32).astype(bf16) * sinm
            qr = qr * (sq * sk).astype(bf16)[None, :]
            ctxs = []
            for b in range(B):
                row = slice(b * SQ, (b + 1) * SQ)
                cols = []
                for hh in range(HC):
                    col = slice(hh * DH, (hh + 1) * DH)
                    s = lax.dot_general(
                        qr[row, col], kr[row, col],
                        (((1,), (1,)), ((), ())),
                        preferred_element_type=f32)
                    e = jnp.exp(s)
                    ctx_u = jnp.dot(e.astype(bf16), vb[row, col],
                                    preferred_element_type=f32)
                    r = 1.0 / jnp.sum(e, axis=-1, keepdims=True)
                    cols.append(ctx_u * r)
                ctxs.append(jnp.concatenate(cols, axis=1))
            ctx = jnp.concatenate(ctxs, axis=0).astype(bf16)
            ctx = ctx * so.astype(bf16)[None, :]
            contrib = jnp.dot(ctx, wo_i.astype(bf16),
                              preferred_element_type=f32)
            for b in range(B):
                rows = contrib[b * SQ:(b + 1) * SQ, :]
                if first:
                    out_ref[b] = rows
                else:
                    out_ref[b] = out_ref[b] + rows

        if _COMPUTE_ONLY:
            for c in range(4):
                stage_comp(c)
            compute(0, first=True)
            for _ in range(7):
                compute(0)
            return

        def mk(src_slot, dst_slot, c, dst_dev, s_sem, r_sem):
            if c < 3:
                src, dst = wbuf.at[src_slot, c], wbuf.at[dst_slot, c]
            else:
                src, dst = obuf.at[src_slot], obuf.at[dst_slot]
            return pltpu.make_async_remote_copy(
                src_ref=src, dst_ref=dst, send_sem=s_sem, recv_sem=r_sem,
                device_id=(dst_dev,), device_id_type=pl.DeviceIdType.MESH)

        sent = []

        def send(src_slot, dst_slot, c, dst_dev, s_sems, r_sems, r, k):
            d = mk(src_slot, dst_slot, c, dst_dev,
                   s_sems.at[r, k], r_sems.at[r, k])
            d.start()
            sent.append(d)

        def recv(dst_slot, c, src_dev, r_sems, r, k):
            mk(dst_slot, dst_slot, c, src_dev,
               s_z.at[0, 0], r_sems.at[r, k]).wait_recv()

        def send_sc(src_sl, dst_sl, c0, c1, dev, s_sems, r_sems, g):
            d = pltpu.make_async_remote_copy(
                src_ref=scb.at[src_sl, c0:c1],
                dst_ref=scb.at[dst_sl, c0:c1],
                send_sem=s_sems.at[g], recv_sem=r_sems.at[g],
                device_id=(dev,), device_id_type=pl.DeviceIdType.MESH)
            d.start()
            sent.append(d)

        def recv_sc(dst_sl, c0, c1, src_dev, r_sems, g):
            pltpu.make_async_remote_copy(
                src_ref=scb.at[dst_sl, c0:c1],
                dst_ref=scb.at[dst_sl, c0:c1],
                send_sem=t_z.at[0], recv_sem=r_sems.at[g],
                device_id=(src_dev,),
                device_id_type=pl.DeviceIdType.MESH).wait_recv()

        S_R, S_L, S_Z = s_r, s_l, s_z
        R_L, R_R, R_Z = r_l, r_r, r_z

        for c in range(4):
            stage_comp(c)
            send(0, 4, c, partner, S_Z, R_Z, 0, c)
            send(0, 3, c, right, S_R, R_L, 0, c)
            send(0, 1, c, left, S_L, R_R, 0, c)
        send_sc(0, 4, 0, 4, partner, t_z, u_z, 0)
        send_sc(0, 3, 0, 4, right, t_r, u_l, 0)
        send_sc(0, 1, 0, 4, left, t_l, u_r, 0)
        compute(0, first=True)

        for c in range(4):
            recv(4, c, partner, R_Z, 0, c)
        recv_sc(4, 0, 4, partner, u_z, 0)
        send(4, 7, 0, right, S_R, R_L, 1, 0)
        send(4, 7, 1, right, S_R, R_L, 1, 1)
        send_sc(4, 7, 0, 2, right, t_r, u_l, 1)
        send(4, 5, 2, left, S_L, R_R, 1, 0)
        send(4, 5, 3, left, S_L, R_R, 1, 1)
        send_sc(4, 5, 2, 4, left, t_l, u_r, 1)
        for c in range(4):
            recv(3, c, left, R_L, 0, c)
        recv_sc(3, 0, 4, left, u_l, 0)
        for c in range(4):
            recv(1, c, right, R_R, 0, c)
        recv_sc(1, 0, 4, right, u_r, 0)
        send(3, 7, 2, partner, S_Z, R_Z, 1, 0)
        send(3, 7, 3, partner, S_Z, R_Z, 1, 1)
        send_sc(3, 7, 2, 4, partner, t_z, u_z, 1)
        send(1, 5, 0, partner, S_Z, R_Z, 1, 2)
        send(1, 5, 1, partner, S_Z, R_Z, 1, 3)
        send_sc(1, 5, 0, 2, partner, t_z, u_z, 2)
        send(3, 2, 0, right, S_R, R_L, 2, 0)
        send(3, 2, 1, right, S_R, R_L, 2, 1)
        send_sc(3, 2, 0, 2, right, t_r, u_l, 2)
        send(1, 2, 2, left, S_L, R_R, 2, 0)
        send(1, 2, 3, left, S_L, R_R, 2, 1)
        send_sc(1, 2, 2, 4, left, t_l, u_r, 2)
        compute(4)
        compute(3)
        compute(1)

        recv(7, 0, left, R_L, 1, 0)
        recv(7, 1, left, R_L, 1, 1)
        recv_sc(7, 0, 2, left, u_l, 1)
        recv(7, 2, partner, R_Z, 1, 0)
        recv(7, 3, partner, R_Z, 1, 1)
        recv_sc(7, 2, 4, partner, u_z, 1)
        send(7, 6, 0, right, S_R, R_L, 2, 2)
        send(7, 6, 1, right, S_R, R_L, 2, 3)
        send_sc(7, 6, 0, 2, right, t_r, u_l, 3)
        compute(7)
        recv(5, 2, right, R_R, 1, 0)
        recv(5, 3, right, R_R, 1, 1)
        recv_sc(5, 2, 4, right, u_r, 1)
        recv(5, 0, partner, R_Z, 1, 2)
        recv(5, 1, partner, R_Z, 1, 3)
        recv_sc(5, 0, 2, partner, u_z, 2)
        send(5, 6, 2, left, S_L, R_R, 2, 2)
        send(5, 6, 3, left, S_L, R_R, 2, 3)
        send_sc(5, 6, 2, 4, left, t_l, u_r, 3)
        compute(5)

        for k, (slot, c) in enumerate([(2, 0), (2, 1)]):
            recv(slot, c, left, R_L, 2, k)
        for k, (slot, c) in enumerate([(2, 2), (2, 3)]):
            recv(slot, c, right, R_R, 2, k)
        recv_sc(2, 0, 2, left, u_l, 2)
        recv_sc(2, 2, 4, right, u_r, 2)
        compute(2)
        for k, (slot, c) in enumerate([(6, 0), (6, 1)], start=2):
            recv(slot, c, left, R_L, 2, k)
        for k, (slot, c) in enumerate([(6, 2), (6, 3)], start=2):
            recv(slot, c, right, R_R, 2, k)
        recv_sc(6, 0, 2, left, u_l, 3)
        recv_sc(6, 2, 4, right, u_r, 3)
        compute(6)

        for d in sent:
            d.wait_send()

    out_shape = jax.ShapeDtypeStruct((B, SQ, D), f32)
    cosm = jnp.asarray(_COS, dtype=bf16)
    sinm = jnp.asarray(_SIN, dtype=bf16)
    rotm = jnp.asarray(_ROT, dtype=bf16)
    return pl.pallas_call(
        body,
        out_shape=out_shape,
        in_specs=[pl.BlockSpec(memory_space=pltpu.VMEM)] * 8,
        out_specs=pl.BlockSpec(memory_space=pltpu.VMEM),
        scratch_shapes=[
            pltpu.VMEM((BSQ, D), bf16),
            pltpu.VMEM((N_DEV, 3, D, CW), i8),
            pltpu.VMEM((N_DEV, CW, D), i8),
            pltpu.VMEM((N_DEV, 4, CW), f32),
            pltpu.SemaphoreType.DMA((3, 4)),
            pltpu.SemaphoreType.DMA((3, 4)),
            pltpu.SemaphoreType.DMA((3, 4)),
            pltpu.SemaphoreType.DMA((3, 4)),
            pltpu.SemaphoreType.DMA((2, 4)),
            pltpu.SemaphoreType.DMA((2, 4)),
            pltpu.SemaphoreType.DMA((4,)),
            pltpu.SemaphoreType.DMA((4,)),
            pltpu.SemaphoreType.DMA((4,)),
            pltpu.SemaphoreType.DMA((4,)),
            pltpu.SemaphoreType.DMA((3,)),
            pltpu.SemaphoreType.DMA((3,)),
        ],
        compiler_params=_CompilerParams(collective_id=0),
    )(x, Wq, Wk, Wv, Wo, cosm, sinm, rotm)


# device time: 45060 ns/iter; 1.0010x vs baseline; 1.0010x over previous
import os as _os

import numpy as np
import jax
import jax.numpy as jnp
from jax import lax
from jax.experimental import pallas as pl
from jax.experimental.pallas import tpu as pltpu

N_DEV = 8
B = 2
SQ = 256
D = 768
HC = 4
DH = 64
CW = HC * DH
BSQ = B * SQ

_sem_signal = getattr(pl, "semaphore_signal", None) or pltpu.semaphore_signal
_sem_wait = getattr(pl, "semaphore_wait", None) or pltpu.semaphore_wait
_CompilerParams = getattr(pltpu, "CompilerParams", None) or getattr(
    pltpu, "TPUCompilerParams"
)

_COMPUTE_ONLY = _os.environ.get("SCBAND_COMPUTE_ONLY") == "1"


def _consts():
    inv = 1.0 / (10000.0 ** (np.arange(0, DH, 2) / DH))
    pos = np.arange(SQ)[:, None] * inv[None, :]
    cos = np.repeat(np.cos(pos), 2, axis=-1)
    sin = np.repeat(np.sin(pos), 2, axis=-1)
    a = np.sqrt(0.125)
    cosm = np.tile(cos * a, (B, HC)).astype(np.float32)
    sinm = np.tile(sin * a, (B, HC)).astype(np.float32)
    r = np.zeros((DH, DH), np.float32)
    for i in range(0, DH, 2):
        r[i + 1, i] = -1.0
        r[i, i + 1] = 1.0
    rot = np.kron(np.eye(HC, dtype=np.float32), r)
    return cosm, sinm, rot


_COS, _SIN, _ROT = _consts()


def kernel(x, Wq, Wk, Wv, Wo):
    bf16 = jnp.bfloat16
    f32 = jnp.float32
    i8 = jnp.int8

    def body(x_ref, wq_ref, wk_ref, wv_ref, wo_ref, cos_ref, sin_ref,
             rot_ref, out_ref, xb, wbuf, obuf, scb,
             s_r, r_l, s_l, r_r, s_z, r_z,
             t_r, u_l, t_l, u_r, t_z, u_z):
        me = lax.axis_index("i")
        base = (me // 4) * 4
        pp = me - base
        right = base + lax.rem(pp + 1, 4)
        left = base + lax.rem(pp + 3, 4)
        partner = lax.rem(me + 4, N_DEV)

        barrier = pltpu.get_barrier_semaphore()
        for nbr in (left, right, partner):
            _sem_signal(barrier, inc=1, device_id=(nbr,),
                        device_id_type=pl.DeviceIdType.MESH)
        _sem_wait(barrier, 3)

        cosm = cos_ref[...]
        sinm = sin_ref[...]
        rotm = rot_ref[...]

        xb[0:SQ, :] = x_ref[0].astype(bf16)
        xb[SQ:BSQ, :] = x_ref[1].astype(bf16)

        def quant(w, pair):
            a = jnp.max(jnp.abs(w), axis=0, keepdims=True)
            if pair:
                ab = a.astype(bf16)
                swap = jnp.dot(ab, jnp.abs(rotm),
                               preferred_element_type=f32).astype(bf16)
                a = jnp.maximum(ab, swap).astype(f32)
            a = jnp.maximum(a, 1e-20)
            qi = jnp.clip(jnp.round(w * (127.0 / a)),
                          -127.0, 127.0).astype(i8)
            return qi, a * (1.0 / 127.0)

        def stage_comp(c):
            if c < 3:
                ref, pair = ((wq_ref, True), (wk_ref, True),
                             (wv_ref, False))[c]
                qi, sc = quant(ref[...], pair)
                wbuf[0, :, c * CW:(c + 1) * CW] = qi
                scb[0, c, :] = sc[0]
            else:
                w = wo_ref[...]
                a = jnp.maximum(jnp.max(jnp.abs(w), axis=1, keepdims=True),
                                1e-20)
                obuf[0] = jnp.clip(jnp.round(w * (127.0 / a)),
                                   -127.0, 127.0).astype(i8)
                scb[0, 3, :] = a[:, 0] * (1.0 / 127.0)

        def compute(slot, first=False):
            xv = xb[...]
            wmat = wbuf[slot].astype(bf16)
            wo_i = obuf[slot]
            sq = scb[slot, 0, :]
            sk = scb[slot, 1, :]
            sv = scb[slot, 2, :]
            so = scb[slot, 3, :]
            qkv = jnp.dot(xv, wmat, preferred_element_type=f32)
            q = qkv[:, 0:CW].astype(bf16)
            k = qkv[:, CW:2 * CW].astype(bf16)
            vb = qkv[:, 2 * CW:3 * CW].astype(bf16) * sv.astype(bf16)[None, :]
            qr = q * cosm + jnp.dot(
                q, rotm, preferred_element_type=f32).astype(bf16) * sinm
            kr = k * cosm + jnp.dot(
                k, rotm, preferred_element_type=f32).astype(bf16) * sinm
            qr = qr * (sq * sk).astype(bf16)[None, :]
            ctxs = []
            for b in range(B):
                row = slice(b * SQ, (b + 1) * SQ)
                cols = []
                for hh in range(HC):
                    col = slice(hh * DH, (hh + 1) * DH)
                    s = lax.dot_general(
                        qr[row, col], kr[row, col],
                        (((1,), (1,)), ((), ())),
                        preferred_element_type=f32)
                    e = jnp.exp(s)
                    ctx_u = jnp.dot(e.astype(bf16), vb[row, col],
                                    preferred_element_type=f32)
                    r = 1.0 / jnp.sum(e, axis=-1, keepdims=True)
                    cols.append(ctx_u * r)
                ctxs.append(jnp.concatenate(cols, axis=1))
            ctx = jnp.concatenate(ctxs, axis=0).astype(bf16)
            ctx = ctx * so.astype(bf16)[None, :]
            contrib = jnp.dot(ctx, wo_i.astype(bf16),
                              preferred_element_type=f32)
            for b in range(B):
                rows = contrib[b * SQ:(b + 1) * SQ, :]
                if first:
                    out_ref[b] = rows
                else:
                    out_ref[b] = out_ref[b] + rows

        if _COMPUTE_ONLY:
            for c in range(4):
                stage_comp(c)
            compute(0, first=True)
            for _ in range(7):
                compute(0)
            return

        def mk(src_slot, dst_slot, c, dst_dev, s_sem, r_sem):
            if c < 3:
                cols = slice(c * CW, (c + 1) * CW)
                src = wbuf.at[src_slot, :, cols]
                dst = wbuf.at[dst_slot, :, cols]
            else:
                src, dst = obuf.at[src_slot], obuf.at[dst_slot]
            return pltpu.make_async_remote_copy(
                src_ref=src, dst_ref=dst, send_sem=s_sem, recv_sem=r_sem,
                device_id=(dst_dev,), device_id_type=pl.DeviceIdType.MESH)

        sent = []

        def send(src_slot, dst_slot, c, dst_dev, s_sems, r_sems, r, k):
            d = mk(src_slot, dst_slot, c, dst_dev,
                   s_sems.at[r, k], r_sems.at[r, k])
            d.start()
            sent.append(d)

        def recv(dst_slot, c, src_dev, r_sems, r, k):
            mk(dst_slot, dst_slot, c, src_dev,
               s_z.at[0, 0], r_sems.at[r, k]).wait_recv()

        def send_sc(src_sl, dst_sl, c0, c1, dev, s_sems, r_sems, g):
            d = pltpu.make_async_remote_copy(
                src_ref=scb.at[src_sl, c0:c1],
                dst_ref=scb.at[dst_sl, c0:c1],
                send_sem=s_sems.at[g], recv_sem=r_sems.at[g],
                device_id=(dev,), device_id_type=pl.DeviceIdType.MESH)
            d.start()
            sent.append(d)

        def recv_sc(dst_sl, c0, c1, src_dev, r_sems, g):
            pltpu.make_async_remote_copy(
                src_ref=scb.at[dst_sl, c0:c1],
                dst_ref=scb.at[dst_sl, c0:c1],
                send_sem=t_z.at[0], recv_sem=r_sems.at[g],
                device_id=(src_dev,),
                device_id_type=pl.DeviceIdType.MESH).wait_recv()

        S_R, S_L, S_Z = s_r, s_l, s_z
        R_L, R_R, R_Z = r_l, r_r, r_z

        for c in range(4):
            stage_comp(c)
            send(0, 4, c, partner, S_Z, R_Z, 0, c)
            send(0, 3, c, right, S_R, R_L, 0, c)
            send(0, 1, c, left, S_L, R_R, 0, c)
        send_sc(0, 4, 0, 4, partner, t_z, u_z, 0)
        send_sc(0, 3, 0, 4, right, t_r, u_l, 0)
        send_sc(0, 1, 0, 4, left, t_l, u_r, 0)
        compute(0, first=True)

        for c in range(4):
            recv(4, c, partner, R_Z, 0, c)
        recv_sc(4, 0, 4, partner, u_z, 0)
        send(4, 7, 0, right, S_R, R_L, 1, 0)
        send(4, 7, 1, right, S_R, R_L, 1, 1)
        send_sc(4, 7, 0, 2, right, t_r, u_l, 1)
        send(4, 5, 2, left, S_L, R_R, 1, 0)
        send(4, 5, 3, left, S_L, R_R, 1, 1)
        send_sc(4, 5, 2, 4, left, t_l, u_r, 1)
        for c in range(4):
            recv(3, c, left, R_L, 0, c)
        recv_sc(3, 0, 4, left, u_l, 0)
        for c in range(4):
            recv(1, c, right, R_R, 0, c)
        recv_sc(1, 0, 4, right, u_r, 0)
        send(3, 7, 2, partner, S_Z, R_Z, 1, 0)
        send(3, 7, 3, partner, S_Z, R_Z, 1, 1)
        send_sc(3, 7, 2, 4, partner, t_z, u_z, 1)
        send(1, 5, 0, partner, S_Z, R_Z, 1, 2)
        send(1, 5, 1, partner, S_Z, R_Z, 1, 3)
        send_sc(1, 5, 0, 2, partner, t_z, u_z, 2)
        send(3, 2, 0, right, S_R, R_L, 2, 0)
        send(3, 2, 1, right, S_R, R_L, 2, 1)
        send_sc(3, 2, 0, 2, right, t_r, u_l, 2)
        send(1, 2, 2, left, S_L, R_R, 2, 0)
        send(1, 2, 3, left, S_L, R_R, 2, 1)
        send_sc(1, 2, 2, 4, left, t_l, u_r, 2)
        compute(4)
        compute(3)
        compute(1)

        recv(7, 0, left, R_L, 1, 0)
        recv(7, 1, left, R_L, 1, 1)
        recv_sc(7, 0, 2, left, u_l, 1)
        recv(7, 2, partner, R_Z, 1, 0)
        recv(7, 3, partner, R_Z, 1, 1)
        recv_sc(7, 2, 4, partner, u_z, 1)
        send(7, 6, 0, right, S_R, R_L, 2, 2)
        send(7, 6, 1, right, S_R, R_L, 2, 3)
        send_sc(7, 6, 0, 2, right, t_r, u_l, 3)
        compute(7)
        recv(5, 2, right, R_R, 1, 0)
        recv(5, 3, right, R_R, 1, 1)
        recv_sc(5, 2, 4, right, u_r, 1)
        recv(5, 0, partner, R_Z, 1, 2)
        recv(5, 1, partner, R_Z, 1, 3)
        recv_sc(5, 0, 2, partner, u_z, 2)
        send(5, 6, 2, left, S_L, R_R, 2, 2)
        send(5, 6, 3, left, S_L, R_R, 2, 3)
        send_sc(5, 6, 2, 4, left, t_l, u_r, 3)
        compute(5)

        for k, (slot, c) in enumerate([(2, 0), (2, 1)]):
            recv(slot, c, left, R_L, 2, k)
        for k, (slot, c) in enumerate([(2, 2), (2, 3)]):
            recv(slot, c, right, R_R, 2, k)
        recv_sc(2, 0, 2, left, u_l, 2)
        recv_sc(2, 2, 4, right, u_r, 2)
        compute(2)
        for k, (slot, c) in enumerate([(6, 0), (6, 1)], start=2):
            recv(slot, c, left, R_L, 2, k)
        for k, (slot, c) in enumerate([(6, 2), (6, 3)], start=2):
            recv(slot, c, right, R_R, 2, k)
        recv_sc(6, 0, 2, left, u_l, 3)
        recv_sc(6, 2, 4, right, u_r, 3)
        compute(6)

        for d in sent:
            d.wait_send()

    out_shape = jax.ShapeDtypeStruct((B, SQ, D), f32)
    cosm = jnp.asarray(_COS, dtype=bf16)
    sinm = jnp.asarray(_SIN, dtype=bf16)
    rotm = jnp.asarray(_ROT, dtype=bf16)
    return pl.pallas_call(
        body,
        out_shape=out_shape,
        in_specs=[pl.BlockSpec(memory_space=pltpu.VMEM)] * 8,
        out_specs=pl.BlockSpec(memory_space=pltpu.VMEM),
        scratch_shapes=[
            pltpu.VMEM((BSQ, D), bf16),
            pltpu.VMEM((N_DEV, D, 3 * CW), i8),
            pltpu.VMEM((N_DEV, CW, D), i8),
            pltpu.VMEM((N_DEV, 4, CW), f32),
            pltpu.SemaphoreType.DMA((3, 4)),
            pltpu.SemaphoreType.DMA((3, 4)),
            pltpu.SemaphoreType.DMA((3, 4)),
            pltpu.SemaphoreType.DMA((3, 4)),
            pltpu.SemaphoreType.DMA((2, 4)),
            pltpu.SemaphoreType.DMA((2, 4)),
            pltpu.SemaphoreType.DMA((4,)),
            pltpu.SemaphoreType.DMA((4,)),
            pltpu.SemaphoreType.DMA((4,)),
            pltpu.SemaphoreType.DMA((4,)),
            pltpu.SemaphoreType.DMA((3,)),
            pltpu.SemaphoreType.DMA((3,)),
        ],
        compiler_params=_CompilerParams(collective_id=0),
    )(x, Wq, Wk, Wv, Wo, cosm, sinm, rotm)
